# baseline (speedup 1.0000x reference)
import functools
import jax
import jax.numpy as jnp
from jax import lax
from jax.experimental import pallas as pl
from jax.experimental.pallas import tpu as pltpu
from jax.experimental.pallas import tpu_sc as plsc

VOCAB = 100000
EMBED = 64
BATCH = 1024

_info = plsc.get_sparse_core_info()
_NC = _info.num_cores
_NS = _info.num_subcores
_NW = _NC * _NS
_BPW = BATCH // _NW


def _sc_gather(table, idx):
    mesh = plsc.VectorSubcoreMesh(core_axis_name="c", subcore_axis_name="s")

    @functools.partial(
        pl.kernel,
        mesh=mesh,
        out_type=jax.ShapeDtypeStruct((BATCH, EMBED), jnp.float32),
        scratch_types=[
            pltpu.VMEM((_BPW,), jnp.int32),
            pltpu.VMEM((_BPW, EMBED), jnp.float32),
            pltpu.SemaphoreType.DMA,
        ],
        compiler_params=pltpu.CompilerParams(use_tc_tiling_on_sc=False),
    )
    def gather_kernel(table_hbm, idx_hbm, out_hbm, idx_v, rows_v, sem):
        wid = lax.axis_index("s") * _NC + lax.axis_index("c")
        base = wid * _BPW
        pltpu.sync_copy(idx_hbm.at[pl.ds(base, _BPW)], idx_v)
        pltpu.async_copy(table_hbm.at[idx_v], rows_v, sem).wait()
        pltpu.sync_copy(rows_v, out_hbm.at[pl.ds(base, _BPW)])

    return gather_kernel(table, idx)


_TILE_V = 2048
_NT = (VOCAB + _TILE_V - 1) // _TILE_V


def _tc_project(emb, W, b2d):
    def mm_kernel(emb_ref, w_ref, b_ref, out_ref):
        out_ref[...] = (
            jnp.dot(emb_ref[...], w_ref[...], preferred_element_type=jnp.float32)
            + b_ref[...]
        )

    return pl.pallas_call(
        mm_kernel,
        grid=(_NT,),
        in_specs=[
            pl.BlockSpec((BATCH, EMBED), lambda j: (0, 0)),
            pl.BlockSpec((EMBED, _TILE_V), lambda j: (0, j)),
            pl.BlockSpec((1, _TILE_V), lambda j: (0, j)),
        ],
        out_specs=pl.BlockSpec((BATCH, _TILE_V), lambda j: (0, j)),
        out_shape=jax.ShapeDtypeStruct((BATCH, VOCAB), jnp.float32),
    )(emb, W, b2d)


def kernel(input, table, W, b):
    idx = input.astype(jnp.int32)
    emb = _sc_gather(table, idx)
    return _tc_project(emb, W, b.reshape(1, VOCAB))


# speedup vs baseline: 1.0478x; 1.0478x over previous
import functools
import jax
import jax.numpy as jnp
from jax import lax
from jax.experimental import pallas as pl
from jax.experimental.pallas import tpu as pltpu
from jax.experimental.pallas import tpu_sc as plsc

VOCAB = 100000
EMBED = 64
BATCH = 1024

_info = plsc.get_sparse_core_info()
_NC = _info.num_cores
_NS = _info.num_subcores
_NW = _NC * _NS
_BPW = BATCH // _NW


def _sc_gather(table, idx):
    mesh = plsc.VectorSubcoreMesh(core_axis_name="c", subcore_axis_name="s")

    @functools.partial(
        pl.kernel,
        mesh=mesh,
        out_type=jax.ShapeDtypeStruct((BATCH, EMBED), jnp.float32),
        scratch_types=[
            pltpu.VMEM((_BPW,), jnp.int32),
            pltpu.VMEM((_BPW, EMBED), jnp.float32),
            pltpu.SemaphoreType.DMA,
        ],
        compiler_params=pltpu.CompilerParams(use_tc_tiling_on_sc=False),
    )
    def gather_kernel(table_hbm, idx_hbm, out_hbm, idx_v, rows_v, sem):
        wid = lax.axis_index("s") * _NC + lax.axis_index("c")
        base = wid * _BPW
        pltpu.sync_copy(idx_hbm.at[pl.ds(base, _BPW)], idx_v)
        pltpu.async_copy(table_hbm.at[idx_v], rows_v, sem).wait()
        pltpu.sync_copy(rows_v, out_hbm.at[pl.ds(base, _BPW)])

    return gather_kernel(table, idx)


_TILE_V = 2048
_NT = (VOCAB + _TILE_V - 1) // _TILE_V


def _tc_project(emb, W, b2d):
    def mm_kernel(emb_ref, w_ref, b_ref, out_ref):
        out_ref[...] = (
            jnp.dot(emb_ref[...], w_ref[...], preferred_element_type=jnp.float32)
            + b_ref[...]
        )

    return pl.pallas_call(
        mm_kernel,
        grid=(_NT,),
        in_specs=[
            pl.BlockSpec((BATCH, EMBED), lambda j: (0, 0)),
            pl.BlockSpec((EMBED, _TILE_V), lambda j: (0, j)),
            pl.BlockSpec((1, _TILE_V), lambda j: (0, j)),
        ],
        out_specs=pl.BlockSpec((BATCH, _TILE_V), lambda j: (0, j)),
        out_shape=jax.ShapeDtypeStruct((BATCH, VOCAB), jnp.float32),
    )(emb, W, b2d)


def kernel(input, table, W, b):
    idx = input.astype(jnp.int32)
    emb = jnp.take(table, idx, axis=0)
    return _tc_project(emb, W, b.reshape(1, VOCAB))
